# baseline (speedup 1.0000x reference)
import functools

import jax
import jax.numpy as jnp
from jax import lax
from jax.experimental import pallas as pl
from jax.experimental.pallas import tpu as pltpu


def _schedule(n):
    rem = n
    head, tail = [], []
    for r in (2, 6):
        if rem >= r + 16:
            head.append(r)
            rem -= r
    for r in (2, 6):
        if rem >= r + 16:
            tail.append(r)
            rem -= r
    mid = []
    while rem > 16:
        mid.append(16)
        rem -= 16
    if rem:
        mid.append(rem)
    return head + mid + tail[::-1]


def _write_milestones(sched):
    marks, acc = [], 0
    for i, c in enumerate(sched):
        acc += c
        if acc >= 24 or i == len(sched) - 1:
            marks.append(i)
            acc = 0
    if marks[-1] != len(sched) - 1:
        marks.append(len(sched) - 1)
    return marks


def _pipeline_kernel(sched, x1_hbm, x2_hbm, v2_hbm,
                     x1_buf, x2_buf, out_buf, s_in, s_out):
    n_chunks = len(sched)
    offs = [0]
    for c in sched:
        offs.append(offs[-1] + c)
    per_core = offs[-1]
    base = pl.program_id(0) * per_core
    marks = _write_milestones(sched)

    def in_copies(i):
        c = sched[i]
        src = pl.ds(base + offs[i], c)
        dst = pl.ds(offs[i], c)
        return (
            pltpu.make_async_copy(x1_hbm.at[src], x1_buf.at[dst], s_in.at[i]),
            pltpu.make_async_copy(x2_hbm.at[src], x2_buf.at[dst], s_in.at[i]),
        )

    def out_copy(m):
        lo = 0 if m == 0 else offs[marks[m - 1] + 1]
        hi = offs[marks[m] + 1]
        return pltpu.make_async_copy(out_buf.at[pl.ds(lo, hi - lo)],
                                     v2_hbm.at[pl.ds(base + lo, hi - lo)],
                                     s_out.at[m])

    for i in range(n_chunks):
        for cp in in_copies(i):
            cp.start()

    for i in range(n_chunks):
        c = sched[i]
        for cp in in_copies(i):
            cp.wait()

        def body(j, carry):
            a = x1_buf[offs[i] + j].astype(jnp.bfloat16)
            b = x2_buf[offs[i] + j].astype(jnp.bfloat16)
            v1 = jnp.dot(a, b, preferred_element_type=jnp.float32)
            out_buf[offs[i] + j] = jnp.dot(v1.astype(jnp.bfloat16), a,
                                           preferred_element_type=jnp.float32)
            return carry

        lax.fori_loop(0, c, body, 0, unroll=min(c, 4))
        if i in marks:
            out_copy(marks.index(i)).start()

    for m in range(len(marks)):
        out_copy(m).wait()


def kernel(x1, x2):
    B, D, D2 = x1.shape
    assert D == D2 and x2.shape == (B, D, D)
    assert B % 2 == 0

    per_core = B // 2
    sched = _schedule(per_core)
    n_chunks = len(sched)
    n_marks = len(_write_milestones(sched))

    itemsize = jnp.dtype(x1.dtype).itemsize
    cost = pl.CostEstimate(
        flops=4 * B * D * D * D,
        transcendentals=0,
        bytes_accessed=3 * B * D * D * itemsize,
    )

    return pl.pallas_call(
        functools.partial(_pipeline_kernel, tuple(sched)),
        out_shape=jax.ShapeDtypeStruct((B, D, D), x1.dtype),
        grid=(2,),
        in_specs=[
            pl.BlockSpec(memory_space=pl.ANY),
            pl.BlockSpec(memory_space=pl.ANY),
        ],
        out_specs=pl.BlockSpec(memory_space=pl.ANY),
        scratch_shapes=[
            pltpu.VMEM((per_core, D, D), x1.dtype),
            pltpu.VMEM((per_core, D, D), x2.dtype),
            pltpu.VMEM((per_core, D, D), x1.dtype),
            pltpu.SemaphoreType.DMA((n_chunks,)),
            pltpu.SemaphoreType.DMA((n_marks,)),
        ],
        compiler_params=pltpu.CompilerParams(
            dimension_semantics=("parallel",),
            vmem_limit_bytes=60 << 20,
        ),
        cost_estimate=cost,
    )(x1, x2)


# speedup vs baseline: 1.0709x; 1.0709x over previous
import functools

import jax
import jax.numpy as jnp
from jax import lax
from jax.experimental import pallas as pl
from jax.experimental.pallas import tpu as pltpu

_WG = 8
_RING = 4


def _schedule(n):
    rem = n
    head, tail = [], []
    for r in (2, 6):
        if rem >= r + 16:
            head.append(r)
            rem -= r
    for r in (2, 6):
        if rem >= r + 16:
            tail.append(r)
            rem -= r
    mid = []
    while rem > 16:
        mid.append(16)
        rem -= 16
    if rem:
        mid.append(rem)
    return head + mid + tail[::-1]


def _groups(sched):
    offs = [0]
    for c in sched:
        offs.append(offs[-1] + c)
    out = []
    for i, c in enumerate(sched):
        o = 0
        while o < c:
            gc = min(_WG, c - o)
            out.append((i, offs[i] + o, gc))
            o += gc
    return out


def _pipeline_kernel(sched, x1_hbm, x2_hbm, v2_hbm,
                     x1_buf, x2_buf, out_buf, s_in, s_out):
    n_chunks = len(sched)
    offs = [0]
    for c in sched:
        offs.append(offs[-1] + c)
    per_core = offs[-1]
    base = pl.program_id(0) * per_core
    groups = _groups(sched)
    n_groups = len(groups)

    def in_copies(i):
        c = sched[i]
        src = pl.ds(base + offs[i], c)
        dst = pl.ds(offs[i], c)
        return (
            pltpu.make_async_copy(x1_hbm.at[src], x1_buf.at[dst], s_in.at[i]),
            pltpu.make_async_copy(x2_hbm.at[src], x2_buf.at[dst], s_in.at[i]),
        )

    def out_copy(g):
        _, go, gc = groups[g]
        p = g % _RING
        return pltpu.make_async_copy(out_buf.at[p, pl.ds(0, gc)],
                                     v2_hbm.at[pl.ds(base + go, gc)],
                                     s_out.at[p])

    for i in range(n_chunks):
        for cp in in_copies(i):
            cp.start()

    g = 0
    for i in range(n_chunks):
        c = sched[i]
        for cp in in_copies(i):
            cp.wait()
        while g < n_groups and groups[g][0] == i:
            _, go, gc = groups[g]
            p = g % _RING
            if g >= _RING:
                out_copy(g - _RING).wait()

            def body(j, carry, go=go, p=p):
                a = x1_buf[go + j].astype(jnp.bfloat16)
                b = x2_buf[go + j].astype(jnp.bfloat16)
                v1 = jnp.dot(a, b, preferred_element_type=jnp.float32)
                out_buf[p, j] = jnp.dot(v1.astype(jnp.bfloat16), a,
                                        preferred_element_type=jnp.float32)
                return carry

            lax.fori_loop(0, gc, body, 0, unroll=min(gc, 4))
            out_copy(g).start()
            g += 1

    for gg in range(max(0, n_groups - _RING), n_groups):
        out_copy(gg).wait()


def kernel(x1, x2):
    B, D, D2 = x1.shape
    assert D == D2 and x2.shape == (B, D, D)
    assert B % 2 == 0

    per_core = B // 2
    sched = _schedule(per_core)
    n_chunks = len(sched)

    itemsize = jnp.dtype(x1.dtype).itemsize
    cost = pl.CostEstimate(
        flops=4 * B * D * D * D,
        transcendentals=0,
        bytes_accessed=3 * B * D * D * itemsize,
    )

    return pl.pallas_call(
        functools.partial(_pipeline_kernel, tuple(sched)),
        out_shape=jax.ShapeDtypeStruct((B, D, D), x1.dtype),
        grid=(2,),
        in_specs=[
            pl.BlockSpec(memory_space=pl.ANY),
            pl.BlockSpec(memory_space=pl.ANY),
        ],
        out_specs=pl.BlockSpec(memory_space=pl.ANY),
        scratch_shapes=[
            pltpu.VMEM((per_core, D, D), x1.dtype),
            pltpu.VMEM((per_core, D, D), x2.dtype),
            pltpu.VMEM((_RING, _WG, D, D), x1.dtype),
            pltpu.SemaphoreType.DMA((n_chunks,)),
            pltpu.SemaphoreType.DMA((_RING,)),
        ],
        compiler_params=pltpu.CompilerParams(
            dimension_semantics=("parallel",),
            vmem_limit_bytes=58 << 20,
        ),
        cost_estimate=cost,
    )(x1, x2)


# speedup vs baseline: 1.0922x; 1.0199x over previous
import functools

import jax
import jax.numpy as jnp
from jax import lax
from jax.experimental import pallas as pl
from jax.experimental.pallas import tpu as pltpu


def _schedule(n):
    rem = n
    head, tail = [], []
    for r in (2, 6):
        if rem >= r + 16:
            head.append(r)
            rem -= r
    for r in (2, 6):
        if rem >= r + 16:
            tail.append(r)
            rem -= r
    mid = []
    while rem > 16:
        mid.append(16)
        rem -= 16
    if rem:
        mid.append(rem)
    return head + mid + tail[::-1]


def _pipeline_kernel(sched, x1_hbm, x2_hbm, v2_hbm,
                     x1_buf, x2_buf, out_buf, s1, so):
    n_chunks = len(sched)
    offs = [0]
    for c in sched:
        offs.append(offs[-1] + c)
    per_core = offs[-1]
    base = pl.program_id(0) * per_core

    def in_copies(i):
        c = sched[i]
        src = pl.ds(base + offs[i], c)
        dst = pl.ds(offs[i], c)
        return (
            pltpu.make_async_copy(x1_hbm.at[src], x1_buf.at[dst], s1.at[i]),
            pltpu.make_async_copy(x2_hbm.at[src], x2_buf.at[dst], s1.at[i]),
        )

    def out_copy(i):
        c, p = sched[i], i % 3
        return pltpu.make_async_copy(out_buf.at[p, pl.ds(0, c)],
                                     v2_hbm.at[pl.ds(base + offs[i], c)],
                                     so.at[p])

    for i in range(n_chunks):
        for cp in in_copies(i):
            cp.start()

    for i in range(n_chunks):
        c, p = sched[i], i % 3
        for cp in in_copies(i):
            cp.wait()
        if i >= 3:
            out_copy(i - 3).wait()

        def body(j, carry):
            a = x1_buf[offs[i] + j].astype(jnp.bfloat16)
            b = x2_buf[offs[i] + j].astype(jnp.bfloat16)
            v1 = jnp.dot(a, b, preferred_element_type=jnp.float32)
            out_buf[p, j] = jnp.dot(v1.astype(jnp.bfloat16), a,
                                    preferred_element_type=jnp.float32)
            return carry

        lax.fori_loop(0, c, body, 0, unroll=min(c, 4))
        out_copy(i).start()

    for i in range(max(0, n_chunks - 3), n_chunks):
        out_copy(i).wait()


def kernel(x1, x2):
    B, D, D2 = x1.shape
    assert D == D2 and x2.shape == (B, D, D)
    assert B % 2 == 0

    per_core = B // 2
    sched = _schedule(per_core)
    n_chunks = len(sched)
    cmax = max(sched)

    itemsize = jnp.dtype(x1.dtype).itemsize
    cost = pl.CostEstimate(
        flops=4 * B * D * D * D,
        transcendentals=0,
        bytes_accessed=3 * B * D * D * itemsize,
    )

    return pl.pallas_call(
        functools.partial(_pipeline_kernel, tuple(sched)),
        out_shape=jax.ShapeDtypeStruct((B, D, D), x1.dtype),
        grid=(2,),
        in_specs=[
            pl.BlockSpec(memory_space=pl.ANY),
            pl.BlockSpec(memory_space=pl.ANY),
        ],
        out_specs=pl.BlockSpec(memory_space=pl.ANY),
        scratch_shapes=[
            pltpu.VMEM((per_core, D, D), x1.dtype),
            pltpu.VMEM((per_core, D, D), x2.dtype),
            pltpu.VMEM((3, cmax, D, D), x1.dtype),
            pltpu.SemaphoreType.DMA((n_chunks,)),
            pltpu.SemaphoreType.DMA((3,)),
        ],
        compiler_params=pltpu.CompilerParams(
            dimension_semantics=("parallel",),
            vmem_limit_bytes=58 << 20,
        ),
        cost_estimate=cost,
    )(x1, x2)


# speedup vs baseline: 1.0927x; 1.0004x over previous
import functools

import jax
import jax.numpy as jnp
from jax import lax
from jax.experimental import pallas as pl
from jax.experimental.pallas import tpu as pltpu


def _schedule(n):
    rem = n
    head, tail = [], []
    for r in (2, 6):
        if rem >= r + 12:
            head.append(r)
            rem -= r
    for r in (2, 6):
        if rem >= r + 12:
            tail.append(r)
            rem -= r
    mid = []
    while rem > 12:
        mid.append(12)
        rem -= 12
    if rem:
        mid.append(rem)
    return head + mid + tail[::-1]


def _pipeline_kernel(sched, x1_hbm, x2_hbm, v2_hbm,
                     x1_buf, x2_buf, out_buf, s1, so):
    n_chunks = len(sched)
    offs = [0]
    for c in sched:
        offs.append(offs[-1] + c)
    per_core = offs[-1]
    base = pl.program_id(0) * per_core

    def in_copies(i):
        c = sched[i]
        src = pl.ds(base + offs[i], c)
        dst = pl.ds(offs[i], c)
        return (
            pltpu.make_async_copy(x1_hbm.at[src], x1_buf.at[dst], s1.at[i]),
            pltpu.make_async_copy(x2_hbm.at[src], x2_buf.at[dst], s1.at[i]),
        )

    def out_copy(i):
        c, p = sched[i], i % 3
        return pltpu.make_async_copy(out_buf.at[p, pl.ds(0, c)],
                                     v2_hbm.at[pl.ds(base + offs[i], c)],
                                     so.at[p])

    for i in range(n_chunks):
        for cp in in_copies(i):
            cp.start()

    for i in range(n_chunks):
        c, p = sched[i], i % 3
        for cp in in_copies(i):
            cp.wait()
        if i >= 3:
            out_copy(i - 3).wait()

        def body(j, carry):
            a = x1_buf[offs[i] + j].astype(jnp.bfloat16)
            b = x2_buf[offs[i] + j].astype(jnp.bfloat16)
            v1 = jnp.dot(a, b, preferred_element_type=jnp.float32)
            out_buf[p, j] = jnp.dot(v1.astype(jnp.bfloat16), a,
                                    preferred_element_type=jnp.float32)
            return carry

        lax.fori_loop(0, c, body, 0, unroll=min(c, 4))
        out_copy(i).start()

    for i in range(max(0, n_chunks - 3), n_chunks):
        out_copy(i).wait()


def kernel(x1, x2):
    B, D, D2 = x1.shape
    assert D == D2 and x2.shape == (B, D, D)
    assert B % 2 == 0

    per_core = B // 2
    sched = _schedule(per_core)
    n_chunks = len(sched)
    cmax = max(sched)

    itemsize = jnp.dtype(x1.dtype).itemsize
    cost = pl.CostEstimate(
        flops=4 * B * D * D * D,
        transcendentals=0,
        bytes_accessed=3 * B * D * D * itemsize,
    )

    return pl.pallas_call(
        functools.partial(_pipeline_kernel, tuple(sched)),
        out_shape=jax.ShapeDtypeStruct((B, D, D), x1.dtype),
        grid=(2,),
        in_specs=[
            pl.BlockSpec(memory_space=pl.ANY),
            pl.BlockSpec(memory_space=pl.ANY),
        ],
        out_specs=pl.BlockSpec(memory_space=pl.ANY),
        scratch_shapes=[
            pltpu.VMEM((per_core, D, D), x1.dtype),
            pltpu.VMEM((per_core, D, D), x2.dtype),
            pltpu.VMEM((3, cmax, D, D), x1.dtype),
            pltpu.SemaphoreType.DMA((n_chunks,)),
            pltpu.SemaphoreType.DMA((3,)),
        ],
        compiler_params=pltpu.CompilerParams(
            dimension_semantics=("parallel",),
            vmem_limit_bytes=58 << 20,
        ),
        cost_estimate=cost,
    )(x1, x2)


# speedup vs baseline: 1.4180x; 1.2977x over previous
import functools

import jax
import jax.numpy as jnp
from jax import lax
from jax.experimental import pallas as pl
from jax.experimental.pallas import tpu as pltpu


def _schedule(n):
    rem = n
    head, tail = [], []
    for r in (2, 6):
        if rem >= r + 8:
            head.append(r)
            rem -= r
    for r in (2, 6):
        if rem >= r + 8:
            tail.append(r)
            rem -= r
    mid = []
    while rem > 8:
        mid.append(8)
        rem -= 12
    if rem:
        mid.append(rem)
    return head + mid + tail[::-1]


def _pipeline_kernel(sched, x1_hbm, x2_hbm, v2_hbm,
                     x1_buf, x2_buf, out_buf, s1, so):
    n_chunks = len(sched)
    offs = [0]
    for c in sched:
        offs.append(offs[-1] + c)
    per_core = offs[-1]
    base = pl.program_id(0) * per_core

    def in_copies(i):
        c = sched[i]
        src = pl.ds(base + offs[i], c)
        dst = pl.ds(offs[i], c)
        return (
            pltpu.make_async_copy(x1_hbm.at[src], x1_buf.at[dst], s1.at[i]),
            pltpu.make_async_copy(x2_hbm.at[src], x2_buf.at[dst], s1.at[i]),
        )

    def out_copy(i):
        c, p = sched[i], i % 3
        return pltpu.make_async_copy(out_buf.at[p, pl.ds(0, c)],
                                     v2_hbm.at[pl.ds(base + offs[i], c)],
                                     so.at[p])

    for i in range(n_chunks):
        for cp in in_copies(i):
            cp.start()

    for i in range(n_chunks):
        c, p = sched[i], i % 3
        for cp in in_copies(i):
            cp.wait()
        if i >= 3:
            out_copy(i - 3).wait()

        def body(j, carry):
            a = x1_buf[offs[i] + j].astype(jnp.bfloat16)
            b = x2_buf[offs[i] + j].astype(jnp.bfloat16)
            v1 = jnp.dot(a, b, preferred_element_type=jnp.float32)
            out_buf[p, j] = jnp.dot(v1.astype(jnp.bfloat16), a,
                                    preferred_element_type=jnp.float32)
            return carry

        lax.fori_loop(0, c, body, 0, unroll=min(c, 4))
        out_copy(i).start()

    for i in range(max(0, n_chunks - 3), n_chunks):
        out_copy(i).wait()


def kernel(x1, x2):
    B, D, D2 = x1.shape
    assert D == D2 and x2.shape == (B, D, D)
    assert B % 2 == 0

    per_core = B // 2
    sched = _schedule(per_core)
    n_chunks = len(sched)
    cmax = max(sched)

    itemsize = jnp.dtype(x1.dtype).itemsize
    cost = pl.CostEstimate(
        flops=4 * B * D * D * D,
        transcendentals=0,
        bytes_accessed=3 * B * D * D * itemsize,
    )

    return pl.pallas_call(
        functools.partial(_pipeline_kernel, tuple(sched)),
        out_shape=jax.ShapeDtypeStruct((B, D, D), x1.dtype),
        grid=(2,),
        in_specs=[
            pl.BlockSpec(memory_space=pl.ANY),
            pl.BlockSpec(memory_space=pl.ANY),
        ],
        out_specs=pl.BlockSpec(memory_space=pl.ANY),
        scratch_shapes=[
            pltpu.VMEM((per_core, D, D), x1.dtype),
            pltpu.VMEM((per_core, D, D), x2.dtype),
            pltpu.VMEM((3, cmax, D, D), x1.dtype),
            pltpu.SemaphoreType.DMA((n_chunks,)),
            pltpu.SemaphoreType.DMA((3,)),
        ],
        compiler_params=pltpu.CompilerParams(
            dimension_semantics=("parallel",),
            vmem_limit_bytes=58 << 20,
        ),
        cost_estimate=cost,
    )(x1, x2)


# speedup vs baseline: 1.9100x; 1.3470x over previous
import functools

import jax
import jax.numpy as jnp
from jax import lax
from jax.experimental import pallas as pl
from jax.experimental.pallas import tpu as pltpu


def _schedule(n):
    rem = n
    head, tail = [], []
    for r in (2, 6):
        if rem >= r + 4:
            head.append(r)
            rem -= r
    for r in (2, 6):
        if rem >= r + 4:
            tail.append(r)
            rem -= r
    mid = []
    while rem > 4:
        mid.append(4)
        rem -= 12
    if rem:
        mid.append(rem)
    return head + mid + tail[::-1]


def _pipeline_kernel(sched, x1_hbm, x2_hbm, v2_hbm,
                     x1_buf, x2_buf, out_buf, s1, so):
    n_chunks = len(sched)
    offs = [0]
    for c in sched:
        offs.append(offs[-1] + c)
    per_core = offs[-1]
    base = pl.program_id(0) * per_core

    def in_copies(i):
        c = sched[i]
        src = pl.ds(base + offs[i], c)
        dst = pl.ds(offs[i], c)
        return (
            pltpu.make_async_copy(x1_hbm.at[src], x1_buf.at[dst], s1.at[i]),
            pltpu.make_async_copy(x2_hbm.at[src], x2_buf.at[dst], s1.at[i]),
        )

    def out_copy(i):
        c, p = sched[i], i % 3
        return pltpu.make_async_copy(out_buf.at[p, pl.ds(0, c)],
                                     v2_hbm.at[pl.ds(base + offs[i], c)],
                                     so.at[p])

    for i in range(n_chunks):
        for cp in in_copies(i):
            cp.start()

    for i in range(n_chunks):
        c, p = sched[i], i % 3
        for cp in in_copies(i):
            cp.wait()
        if i >= 3:
            out_copy(i - 3).wait()

        def body(j, carry):
            a = x1_buf[offs[i] + j].astype(jnp.bfloat16)
            b = x2_buf[offs[i] + j].astype(jnp.bfloat16)
            v1 = jnp.dot(a, b, preferred_element_type=jnp.float32)
            out_buf[p, j] = jnp.dot(v1.astype(jnp.bfloat16), a,
                                    preferred_element_type=jnp.float32)
            return carry

        lax.fori_loop(0, c, body, 0, unroll=min(c, 4))
        out_copy(i).start()

    for i in range(max(0, n_chunks - 3), n_chunks):
        out_copy(i).wait()


def kernel(x1, x2):
    B, D, D2 = x1.shape
    assert D == D2 and x2.shape == (B, D, D)
    assert B % 2 == 0

    per_core = B // 2
    sched = _schedule(per_core)
    n_chunks = len(sched)
    cmax = max(sched)

    itemsize = jnp.dtype(x1.dtype).itemsize
    cost = pl.CostEstimate(
        flops=4 * B * D * D * D,
        transcendentals=0,
        bytes_accessed=3 * B * D * D * itemsize,
    )

    return pl.pallas_call(
        functools.partial(_pipeline_kernel, tuple(sched)),
        out_shape=jax.ShapeDtypeStruct((B, D, D), x1.dtype),
        grid=(2,),
        in_specs=[
            pl.BlockSpec(memory_space=pl.ANY),
            pl.BlockSpec(memory_space=pl.ANY),
        ],
        out_specs=pl.BlockSpec(memory_space=pl.ANY),
        scratch_shapes=[
            pltpu.VMEM((per_core, D, D), x1.dtype),
            pltpu.VMEM((per_core, D, D), x2.dtype),
            pltpu.VMEM((3, cmax, D, D), x1.dtype),
            pltpu.SemaphoreType.DMA((n_chunks,)),
            pltpu.SemaphoreType.DMA((3,)),
        ],
        compiler_params=pltpu.CompilerParams(
            dimension_semantics=("parallel",),
            vmem_limit_bytes=58 << 20,
        ),
        cost_estimate=cost,
    )(x1, x2)
